# baseline (speedup 1.0000x reference)
import functools

import jax
import jax.numpy as jnp
from jax.experimental import pallas as pl


def _silu(v):
    return v * jax.lax.logistic(v)


def _layer_body(h_ref, x_ref, xT_ref, we1a_ref, we1b_ref, we1c_ref, be1_ref,
                we2_ref, be2_ref, wx1_ref, bx1_ref, wx2r_ref, bx2_ref,
                wh1a_ref, wh1b_ref, bh1_ref, wh2_ref, bh2_ref,
                oh_ref, ox_ref, *, bi, n, d):
    i = pl.program_id(0)
    r0 = i * bi
    h_all = h_ref[:, :]
    hi = h_ref[pl.ds(r0, bi), :]
    xi = x_ref[pl.ds(r0, bi), :]

    d2 = jnp.zeros((bi, n), jnp.float32)
    for k in range(3):
        dk = xi[:, k:k + 1] - xT_ref[k:k + 1, :]
        d2 = d2 + dk * dk

    ai = jnp.dot(hi, we1a_ref[:, :], preferred_element_type=jnp.float32)
    bj = jnp.dot(h_all, we1b_ref[:, :], preferred_element_type=jnp.float32)
    m0 = (ai[:, None, :] + bj[None, :, :]
          + d2[:, :, None] * we1c_ref[0, :][None, None, :]
          + be1_ref[0, :][None, None, :])
    m1 = _silu(m0).reshape(bi * n, d)
    m = _silu(jnp.dot(m1, we2_ref[:, :], preferred_element_type=jnp.float32)
              + be2_ref[0, :][None, :])

    t = _silu(jnp.dot(m, wx1_ref[:, :], preferred_element_type=jnp.float32)
              + bx1_ref[0, :][None, :])
    c = jnp.sum(t * wx2r_ref[0, :][None, :], axis=1) + bx2_ref[0, 0]
    c2 = c.reshape(bi, n)

    cols = jax.lax.broadcasted_iota(jnp.int32, (bi, n), 1)
    rows = jax.lax.broadcasted_iota(jnp.int32, (bi, n), 0) + r0
    mask = (rows != cols).astype(jnp.float32)

    cm = c2 * mask
    csum = jnp.sum(cm, axis=1, keepdims=True)
    cx = jnp.dot(cm, x_ref[:, :], preferred_element_type=jnp.float32)
    ox_ref[:, :] = xi + (xi * csum - cx) * (1.0 / (n - 1))

    magg = jnp.sum(m.reshape(bi, n, d) * mask[:, :, None], axis=1)
    g = _silu(jnp.dot(hi, wh1a_ref[:, :], preferred_element_type=jnp.float32)
              + jnp.dot(magg, wh1b_ref[:, :], preferred_element_type=jnp.float32)
              + bh1_ref[0, :][None, :])
    hupd = jnp.dot(g, wh2_ref[:, :], preferred_element_type=jnp.float32) \
        + bh2_ref[0, :][None, :]
    oh_ref[:, :] = _silu(hi + hupd)


def _egnn_layer(h, x, We1, be1, We2, be2, Wx1, bx1, Wx2, bx2,
                Wh1, bh1, Wh2, bh2, *, bi, interpret=False):
    n, d = h.shape
    xT = x.T
    we1a = We1[:d]
    we1b = We1[d:2 * d]
    we1c = We1[2 * d:2 * d + 1]
    wx2r = Wx2.T
    bx2m = bx2.reshape(1, 1)
    wh1a = Wh1[:d]
    wh1b = Wh1[d:]

    full = lambda shape: pl.BlockSpec(shape, lambda i: (0, 0))
    body = functools.partial(_layer_body, bi=bi, n=n, d=d)
    return pl.pallas_call(
        body,
        grid=(n // bi,),
        in_specs=[
            full((n, d)),
            full((n, 3)),
            full((3, n)),
            full((d, d)),
            full((d, d)),
            full((1, d)),
            full((1, d)),
            full((d, d)),
            full((1, d)),
            full((d, d)),
            full((1, d)),
            full((1, d)),
            full((1, 1)),
            full((d, d)),
            full((d, d)),
            full((1, d)),
            full((d, d)),
            full((1, d)),
        ],
        out_specs=[
            pl.BlockSpec((bi, d), lambda i: (i, 0)),
            pl.BlockSpec((bi, 3), lambda i: (i, 0)),
        ],
        out_shape=[
            jax.ShapeDtypeStruct((n, d), jnp.float32),
            jax.ShapeDtypeStruct((n, 3), jnp.float32),
        ],
        interpret=interpret,
    )(h, x, xT, we1a, we1b, we1c, be1.reshape(1, d), We2, be2.reshape(1, d),
      Wx1, bx1.reshape(1, d), wx2r, bx2m, wh1a, wh1b, bh1.reshape(1, d),
      Wh2, bh2.reshape(1, d))


def kernel(h, x, We1, be1, We2, be2, Wx1, bx1, Wx2, bx2, Wh1, bh1, Wh2, bh2):
    L = We1.shape[0]
    for l in range(L):
        h, x = _egnn_layer(h, x, We1[l], be1[l], We2[l], be2[l],
                           Wx1[l], bx1[l], Wx2[l], bx2[l],
                           Wh1[l], bh1[l], Wh2[l], bh2[l], bi=16)
    return (h, x)
